# baseline (device time: 120954 ns/iter reference)
import jax
import jax.numpy as jnp
from jax import lax
from jax.experimental import pallas as pl
from jax.experimental.pallas import tpu as pltpu

N_DEV = 16
M_PER = 64
D = 512
H_PER = 1024
GROUP_SIZES = [1, 4, 4, 4, 2, 1]


def kernel(x, Win0, Wout0, Win1, Wout1, Win2, Wout2):
    def body(x_ref, win0, wout0, win1, wout1, win2, wout2, out_ref,
             xg_ref, part_ref, rs_ref, ag_sems, rs_sems, ag_snd, rs_snd):
        my_i = lax.axis_index("i")

        xg_ref[pl.ds(0, M_PER), :] = x_ref[:, :].astype(jnp.bfloat16)

        for layer, (wi, wo) in enumerate(
            [(win0, wout0), (win1, wout1), (win2, wout2)]
        ):
            wi_b = wi[:, :].astype(jnp.bfloat16)
            wo_b = wo[:, :].astype(jnp.bfloat16)
            ag_rdmas = []
            sendq = list(range(N_DEV - 1, 0, -1))

            def issue_ag(d):
                tgt = lax.rem(my_i + d, N_DEV)
                rdma = pltpu.make_async_remote_copy(
                    src_ref=xg_ref.at[pl.ds(0, M_PER), :],
                    dst_ref=xg_ref.at[pl.ds((N_DEV - d) * M_PER, M_PER), :],
                    send_sem=ag_snd.at[d],
                    recv_sem=ag_sems.at[my_i],
                    device_id=(tgt,),
                    device_id_type=pl.DeviceIdType.MESH,
                )
                rdma.start()
                ag_rdmas.append(rdma)

            AG_DEPTH = 3
            for k in range(AG_DEPTH):
                issue_ag(sendq[k])
            next_send = AG_DEPTH

            rs_rdmas = []
            d0 = 0
            for gsz in GROUP_SIZES:
                for d in range(d0, d0 + gsz):
                    if d == 0:
                        continue
                    src_j = lax.rem(my_i + d, N_DEV)
                    recv = pltpu.make_async_remote_copy(
                        src_ref=xg_ref.at[pl.ds(d * M_PER, M_PER), :],
                        dst_ref=xg_ref.at[pl.ds(d * M_PER, M_PER), :],
                        send_sem=ag_snd.at[d],
                        recv_sem=ag_sems.at[src_j],
                        device_id=(src_j,),
                        device_id_type=pl.DeviceIdType.MESH,
                    )
                    recv.wait_recv()
                    if next_send < N_DEV - 1:
                        issue_ag(sendq[next_send])
                        next_send += 1
                rows = pl.ds(d0 * M_PER, gsz * M_PER)
                h = jnp.maximum(
                    jnp.dot(xg_ref[rows, :], wi_b,
                            preferred_element_type=jnp.float32),
                    0.0,
                ).astype(jnp.bfloat16)
                part = jnp.dot(h, wo_b, preferred_element_type=jnp.float32)
                if d0 == 0:
                    rs_ref[my_i] = part[:M_PER, :].astype(jnp.bfloat16)
                part_ref[rows, :] = part.astype(jnp.bfloat16)
                for d in range(d0, d0 + gsz):
                    if d == 0:
                        continue
                    tgt = lax.rem(my_i + d, N_DEV)
                    rdma = pltpu.make_async_remote_copy(
                        src_ref=part_ref.at[pl.ds(d * M_PER, M_PER), :],
                        dst_ref=rs_ref.at[my_i],
                        send_sem=rs_snd.at[d],
                        recv_sem=rs_sems.at[my_i],
                        device_id=(tgt,),
                        device_id_type=pl.DeviceIdType.MESH,
                    )
                    rdma.start()
                    rs_rdmas.append(rdma)
                d0 += gsz

            for d in range(N_DEV - 1, 0, -1):
                src_j = lax.rem(my_i + d, N_DEV)
                recv = pltpu.make_async_remote_copy(
                    src_ref=part_ref.at[pl.ds(d * M_PER, M_PER), :],
                    dst_ref=rs_ref.at[src_j],
                    send_sem=rs_snd.at[d],
                    recv_sem=rs_sems.at[src_j],
                    device_id=(src_j,),
                    device_id_type=pl.DeviceIdType.MESH,
                )
                recv.wait_recv()
            new_x = jnp.sum(rs_ref[:, :, :].astype(jnp.float32), axis=0)
            for r in ag_rdmas:
                r.wait_send()
            for r in rs_rdmas:
                r.wait_send()
            if layer < 2:
                xg_ref[pl.ds(0, M_PER), :] = new_x.astype(jnp.bfloat16)
            else:
                out_ref[:, :] = new_x

    return pl.pallas_call(
        body,
        out_shape=jax.ShapeDtypeStruct((M_PER, D), jnp.float32),
        in_specs=[pl.BlockSpec(memory_space=pltpu.VMEM)] * 7,
        out_specs=pl.BlockSpec(memory_space=pltpu.VMEM),
        scratch_shapes=[
            pltpu.VMEM((N_DEV * M_PER, D), jnp.bfloat16),
            pltpu.VMEM((N_DEV * M_PER, D), jnp.bfloat16),
            pltpu.VMEM((N_DEV, M_PER, D), jnp.bfloat16),
            pltpu.SemaphoreType.DMA((N_DEV,)),
            pltpu.SemaphoreType.DMA((N_DEV,)),
            pltpu.SemaphoreType.DMA((N_DEV,)),
            pltpu.SemaphoreType.DMA((N_DEV,)),
        ],
    )(x, Win0, Wout0, Win1, Wout1, Win2, Wout2)


# device time: 94738 ns/iter; 1.2767x vs baseline; 1.2767x over previous
import jax
import jax.numpy as jnp
from jax import lax
from jax.experimental import pallas as pl
from jax.experimental.pallas import tpu as pltpu

N_DEV = 16
M_PER = 64
D = 512
H_PER = 1024
GROUP_SIZES = [1, 4, 4, 4, 2, 1]


def kernel(x, Win0, Wout0, Win1, Wout1, Win2, Wout2):
    def body(x_ref, win0, wout0, win1, wout1, win2, wout2, out_ref,
             xg_ref, part_ref, rs_ref, ag_sems, rs_sems, ag_snd, rs_snd):
        my_i = lax.axis_index("i")

        xg_ref[pl.ds(0, M_PER), :] = x_ref[:, :].astype(jnp.bfloat16)

        for layer, (wi, wo) in enumerate(
            [(win0, wout0), (win1, wout1), (win2, wout2)]
        ):
            wi_b = wi[:, :].astype(jnp.bfloat16)
            wo_b = wo[:, :].astype(jnp.bfloat16)
            ag_rdmas = []
            for d in range(1, N_DEV):
                tgt = lax.rem(my_i + d, N_DEV)
                rdma = pltpu.make_async_remote_copy(
                    src_ref=xg_ref.at[pl.ds(0, M_PER), :],
                    dst_ref=xg_ref.at[pl.ds((N_DEV - d) * M_PER, M_PER), :],
                    send_sem=ag_snd.at[d],
                    recv_sem=ag_sems.at[my_i],
                    device_id=(tgt,),
                    device_id_type=pl.DeviceIdType.MESH,
                )
                rdma.start()
                ag_rdmas.append(rdma)

            rs_rdmas = []
            d0 = 0
            for gsz in GROUP_SIZES:
                for d in range(d0, d0 + gsz):
                    if d == 0:
                        continue
                    src_j = lax.rem(my_i + d, N_DEV)
                    recv = pltpu.make_async_remote_copy(
                        src_ref=xg_ref.at[pl.ds(d * M_PER, M_PER), :],
                        dst_ref=xg_ref.at[pl.ds(d * M_PER, M_PER), :],
                        send_sem=ag_snd.at[d],
                        recv_sem=ag_sems.at[src_j],
                        device_id=(src_j,),
                        device_id_type=pl.DeviceIdType.MESH,
                    )
                    recv.wait_recv()
                rows = pl.ds(d0 * M_PER, gsz * M_PER)
                h = jnp.maximum(
                    jnp.dot(xg_ref[rows, :], wi_b,
                            preferred_element_type=jnp.float32),
                    0.0,
                ).astype(jnp.bfloat16)
                part_ref[rows, :] = jnp.dot(
                    h, wo_b, preferred_element_type=jnp.float32
                ).astype(jnp.bfloat16)
                for d in range(d0, d0 + gsz):
                    if d == 0:
                        rs_ref[my_i] = part_ref[pl.ds(0, M_PER), :]
                        continue
                    tgt = lax.rem(my_i + d, N_DEV)
                    rdma = pltpu.make_async_remote_copy(
                        src_ref=part_ref.at[pl.ds(d * M_PER, M_PER), :],
                        dst_ref=rs_ref.at[my_i],
                        send_sem=rs_snd.at[d],
                        recv_sem=rs_sems.at[my_i],
                        device_id=(tgt,),
                        device_id_type=pl.DeviceIdType.MESH,
                    )
                    rdma.start()
                    rs_rdmas.append(rdma)
                d0 += gsz

            for d in range(1, N_DEV):
                src_j = lax.rem(my_i + d, N_DEV)
                recv = pltpu.make_async_remote_copy(
                    src_ref=part_ref.at[pl.ds(d * M_PER, M_PER), :],
                    dst_ref=rs_ref.at[src_j],
                    send_sem=rs_snd.at[d],
                    recv_sem=rs_sems.at[src_j],
                    device_id=(src_j,),
                    device_id_type=pl.DeviceIdType.MESH,
                )
                recv.wait_recv()
            new_x = jnp.sum(rs_ref[:, :, :].astype(jnp.float32), axis=0)
            for r in ag_rdmas:
                r.wait_send()
            for r in rs_rdmas:
                r.wait_send()
            if layer < 2:
                xg_ref[pl.ds(0, M_PER), :] = new_x.astype(jnp.bfloat16)
            else:
                out_ref[:, :] = new_x

    return pl.pallas_call(
        body,
        out_shape=jax.ShapeDtypeStruct((M_PER, D), jnp.float32),
        in_specs=[pl.BlockSpec(memory_space=pltpu.VMEM)] * 7,
        out_specs=pl.BlockSpec(memory_space=pltpu.VMEM),
        scratch_shapes=[
            pltpu.VMEM((N_DEV * M_PER, D), jnp.bfloat16),
            pltpu.VMEM((N_DEV * M_PER, D), jnp.bfloat16),
            pltpu.VMEM((N_DEV, M_PER, D), jnp.bfloat16),
            pltpu.SemaphoreType.DMA((N_DEV,)),
            pltpu.SemaphoreType.DMA((N_DEV,)),
            pltpu.SemaphoreType.DMA((N_DEV,)),
            pltpu.SemaphoreType.DMA((N_DEV,)),
        ],
    )(x, Win0, Wout0, Win1, Wout1, Win2, Wout2)


# device time: 94216 ns/iter; 1.2838x vs baseline; 1.0055x over previous
import jax
import jax.numpy as jnp
from jax import lax
from jax.experimental import pallas as pl
from jax.experimental.pallas import tpu as pltpu

N_DEV = 16
M_PER = 64
D = 512
H_PER = 1024
GROUP_SIZES = [1, 4, 4, 4, 2, 1]


def kernel(x, Win0, Wout0, Win1, Wout1, Win2, Wout2):
    def body(x_ref, win0, wout0, win1, wout1, win2, wout2, out_ref,
             xg_ref, part_ref, rs_ref, win_v, wout_v,
             ag_sems, rs_sems, ag_snd, rs_snd, w_sems):
        my_i = lax.axis_index("i")

        w_copies = []
        for k, (src, dst) in enumerate([
            (win0, win_v.at[0]), (wout0, wout_v.at[0]),
            (win1, win_v.at[1]), (wout1, wout_v.at[1]),
            (win2, win_v.at[2]), (wout2, wout_v.at[2]),
        ]):
            cp = pltpu.make_async_copy(src, dst, w_sems.at[k])
            cp.start()
            w_copies.append(cp)

        xg_ref[pl.ds(0, M_PER), :] = x_ref[:, :].astype(jnp.bfloat16)

        for layer in range(3):
            ag_rdmas = []
            for d in range(1, N_DEV):
                tgt = lax.rem(my_i + d, N_DEV)
                rdma = pltpu.make_async_remote_copy(
                    src_ref=xg_ref.at[pl.ds(0, M_PER), :],
                    dst_ref=xg_ref.at[pl.ds((N_DEV - d) * M_PER, M_PER), :],
                    send_sem=ag_snd.at[d],
                    recv_sem=ag_sems.at[my_i],
                    device_id=(tgt,),
                    device_id_type=pl.DeviceIdType.MESH,
                )
                rdma.start()
                ag_rdmas.append(rdma)

            w_copies[2 * layer].wait()
            w_copies[2 * layer + 1].wait()
            wi_b = win_v[layer].astype(jnp.bfloat16)
            wo_b = wout_v[layer].astype(jnp.bfloat16)

            rs_rdmas = []
            d0 = 0
            for gsz in GROUP_SIZES:
                for d in range(d0, d0 + gsz):
                    if d == 0:
                        continue
                    src_j = lax.rem(my_i + d, N_DEV)
                    recv = pltpu.make_async_remote_copy(
                        src_ref=xg_ref.at[pl.ds(d * M_PER, M_PER), :],
                        dst_ref=xg_ref.at[pl.ds(d * M_PER, M_PER), :],
                        send_sem=ag_snd.at[d],
                        recv_sem=ag_sems.at[src_j],
                        device_id=(src_j,),
                        device_id_type=pl.DeviceIdType.MESH,
                    )
                    recv.wait_recv()
                rows = pl.ds(d0 * M_PER, gsz * M_PER)
                h = jnp.maximum(
                    jnp.dot(xg_ref[rows, :], wi_b,
                            preferred_element_type=jnp.float32),
                    0.0,
                ).astype(jnp.bfloat16)
                part_ref[rows, :] = jnp.dot(
                    h, wo_b, preferred_element_type=jnp.float32
                ).astype(jnp.bfloat16)
                for d in range(d0, d0 + gsz):
                    if d == 0:
                        rs_ref[my_i] = part_ref[pl.ds(0, M_PER), :]
                        continue
                    tgt = lax.rem(my_i + d, N_DEV)
                    rdma = pltpu.make_async_remote_copy(
                        src_ref=part_ref.at[pl.ds(d * M_PER, M_PER), :],
                        dst_ref=rs_ref.at[my_i],
                        send_sem=rs_snd.at[d],
                        recv_sem=rs_sems.at[my_i],
                        device_id=(tgt,),
                        device_id_type=pl.DeviceIdType.MESH,
                    )
                    rdma.start()
                    rs_rdmas.append(rdma)
                d0 += gsz

            for d in range(1, N_DEV):
                src_j = lax.rem(my_i + d, N_DEV)
                recv = pltpu.make_async_remote_copy(
                    src_ref=part_ref.at[pl.ds(d * M_PER, M_PER), :],
                    dst_ref=rs_ref.at[src_j],
                    send_sem=rs_snd.at[d],
                    recv_sem=rs_sems.at[src_j],
                    device_id=(src_j,),
                    device_id_type=pl.DeviceIdType.MESH,
                )
                recv.wait_recv()
            new_x = jnp.sum(rs_ref[:, :, :].astype(jnp.float32), axis=0)
            for r in ag_rdmas:
                r.wait_send()
            for r in rs_rdmas:
                r.wait_send()
            if layer < 2:
                xg_ref[pl.ds(0, M_PER), :] = new_x.astype(jnp.bfloat16)
            else:
                out_ref[:, :] = new_x

    return pl.pallas_call(
        body,
        out_shape=jax.ShapeDtypeStruct((M_PER, D), jnp.float32),
        in_specs=[pl.BlockSpec(memory_space=pltpu.VMEM)]
        + [pl.BlockSpec(memory_space=pl.ANY)] * 6,
        out_specs=pl.BlockSpec(memory_space=pltpu.VMEM),
        scratch_shapes=[
            pltpu.VMEM((N_DEV * M_PER, D), jnp.bfloat16),
            pltpu.VMEM((N_DEV * M_PER, D), jnp.bfloat16),
            pltpu.VMEM((N_DEV, M_PER, D), jnp.bfloat16),
            pltpu.VMEM((3, D, H_PER), jnp.float32),
            pltpu.VMEM((3, H_PER, D), jnp.float32),
            pltpu.SemaphoreType.DMA((N_DEV,)),
            pltpu.SemaphoreType.DMA((N_DEV,)),
            pltpu.SemaphoreType.DMA((N_DEV,)),
            pltpu.SemaphoreType.DMA((N_DEV,)),
            pltpu.SemaphoreType.DMA((6,)),
        ],
    )(x, Win0, Wout0, Win1, Wout1, Win2, Wout2)
